# baseline (speedup 1.0000x reference)
import functools

import jax
import jax.numpy as jnp
from jax import lax
from jax.experimental import pallas as pl
from jax.experimental.pallas import tpu as pltpu
from jax.experimental.pallas import tpu_sc as plsc

BATCH = 16384
EMBD_DIM = 32
NUM_CORES = 2
NUM_SUBCORES = 16
NUM_WORKERS = NUM_CORES * NUM_SUBCORES
B_PER_W = BATCH // NUM_WORKERS
LANES = 16


def _lookup_body(idx_hbm, table_hbm, out_hbm, tab_v, out_v, idx_v, sem):
    wid = lax.axis_index("s") * NUM_CORES + lax.axis_index("c")
    pltpu.sync_copy(table_hbm, tab_v)
    pltpu.sync_copy(
        idx_hbm.at[pl.ds(wid * B_PER_W, B_PER_W)], idx_v.at[pl.ds(0, B_PER_W)]
    )

    t0a = tab_v[pl.ds(0, LANES)]
    t0b = tab_v[pl.ds(16, LANES)]
    t1a = tab_v[pl.ds(32, LANES)]
    t1b = tab_v[pl.ds(48, LANES)]
    t2a = tab_v[pl.ds(64, LANES)]
    t2b = tab_v[pl.ds(80, LANES)]
    zero16 = jnp.zeros((LANES,), jnp.int32)

    half = B_PER_W // 4
    copies = []
    for h in range(4):

        @plsc.parallel_loop(h * half, (h + 1) * half, unroll=8)
        def _row(i):
            vi = idx_v[pl.ds(i, LANES)]
            s = vi.at[zero16].get(mode="promise_in_bounds")
            m1 = s == 1
            m2 = s == 2
            out_v[i, pl.ds(0, LANES)] = jnp.where(m1, t1a, jnp.where(m2, t2a, t0a))
            out_v[i, pl.ds(LANES, LANES)] = jnp.where(
                m1, t1b, jnp.where(m2, t2b, t0b)
            )

        copies.append(
            pltpu.async_copy(
                out_v.at[pl.ds(h * half, half)],
                out_hbm.at[pl.ds(wid * B_PER_W + h * half, half)],
                sem,
            )
        )
    for c in copies:
        c.wait()


@jax.jit
def _lookup(idx_flat, table_flat):
    mesh = plsc.VectorSubcoreMesh(core_axis_name="c", subcore_axis_name="s")
    return pl.kernel(
        _lookup_body,
        out_type=jax.ShapeDtypeStruct((BATCH, EMBD_DIM), jnp.float32),
        mesh=mesh,
        compiler_params=pltpu.CompilerParams(needs_layout_passes=False),
        scratch_types=[
            pltpu.VMEM((3 * EMBD_DIM,), jnp.float32),
            pltpu.VMEM((B_PER_W, EMBD_DIM), jnp.float32),
            pltpu.VMEM((B_PER_W + LANES,), jnp.int32),
            pltpu.SemaphoreType.DMA,
        ],
    )(idx_flat, table_flat)


def kernel(bus_type, embd_table):
    idx_flat = bus_type.astype(jnp.int32).reshape(BATCH)
    return _lookup(idx_flat, embd_table.reshape(-1))


# speedup vs baseline: 1.0134x; 1.0134x over previous
import functools

import jax
import jax.numpy as jnp
from jax import lax
from jax.experimental import pallas as pl
from jax.experimental.pallas import tpu as pltpu
from jax.experimental.pallas import tpu_sc as plsc

BATCH = 16384
EMBD_DIM = 32
NUM_CORES = 2
NUM_SUBCORES = 16
NUM_WORKERS = NUM_CORES * NUM_SUBCORES
B_PER_W = BATCH // NUM_WORKERS
LANES = 16


def _lookup_body(idx_hbm, table_hbm, out_hbm, tab_v, out_v, idx_v, sem):
    wid = lax.axis_index("s") * NUM_CORES + lax.axis_index("c")
    pltpu.sync_copy(table_hbm, tab_v)
    pltpu.sync_copy(
        idx_hbm.at[pl.ds(wid * B_PER_W, B_PER_W)], idx_v.at[pl.ds(0, B_PER_W)]
    )

    t0a = tab_v[pl.ds(0, LANES)]
    t0b = tab_v[pl.ds(16, LANES)]
    t1a = tab_v[pl.ds(32, LANES)]
    t1b = tab_v[pl.ds(48, LANES)]
    t2a = tab_v[pl.ds(64, LANES)]
    t2b = tab_v[pl.ds(80, LANES)]
    zero16 = jnp.zeros((LANES,), jnp.int32)

    half = B_PER_W // 2
    copies = []
    for h in range(2):

        @plsc.parallel_loop(h * half, (h + 1) * half, unroll=8)
        def _row(i):
            vi = idx_v[pl.ds(i, LANES)]
            s = vi.at[zero16].get(mode="promise_in_bounds")
            m1 = s == 1
            m2 = s == 2
            out_v[i, pl.ds(0, LANES)] = jnp.where(m1, t1a, jnp.where(m2, t2a, t0a))
            out_v[i, pl.ds(LANES, LANES)] = jnp.where(
                m1, t1b, jnp.where(m2, t2b, t0b)
            )

        copies.append(
            pltpu.async_copy(
                out_v.at[pl.ds(h * half, half)],
                out_hbm.at[pl.ds(wid * B_PER_W + h * half, half)],
                sem,
            )
        )
    for c in copies:
        c.wait()


@jax.jit
def _lookup(idx_flat, table_flat):
    mesh = plsc.VectorSubcoreMesh(core_axis_name="c", subcore_axis_name="s")
    return pl.kernel(
        _lookup_body,
        out_type=jax.ShapeDtypeStruct((BATCH, EMBD_DIM), jnp.float32),
        mesh=mesh,
        compiler_params=pltpu.CompilerParams(needs_layout_passes=False),
        scratch_types=[
            pltpu.VMEM((3 * EMBD_DIM,), jnp.float32),
            pltpu.VMEM((B_PER_W, EMBD_DIM), jnp.float32),
            pltpu.VMEM((B_PER_W + LANES,), jnp.int32),
            pltpu.SemaphoreType.DMA,
        ],
    )(idx_flat, table_flat)


def kernel(bus_type, embd_table):
    idx_flat = bus_type.astype(jnp.int32).reshape(BATCH)
    return _lookup(idx_flat, embd_table.reshape(-1))


# speedup vs baseline: 1.0222x; 1.0088x over previous
import jax
import jax.numpy as jnp
from jax import lax
from jax.experimental import pallas as pl
from jax.experimental.pallas import tpu as pltpu
from jax.experimental.pallas import tpu_sc as plsc

BATCH = 16384
EMBD_DIM = 32
NUM_CORES = 2
NUM_SUBCORES = 16
NUM_WORKERS = NUM_CORES * NUM_SUBCORES
B_PER_W = BATCH // NUM_WORKERS
LANES = 16


def _lookup_body(idx_hbm, table_hbm, out_hbm, tab_v, out_v, idx_v, sem):
    wid = lax.axis_index("s") * NUM_CORES + lax.axis_index("c")
    c_tab = pltpu.async_copy(table_hbm, tab_v, sem)
    c_idx = pltpu.async_copy(
        idx_hbm.at[pl.ds(wid * B_PER_W, B_PER_W)], idx_v.at[pl.ds(0, B_PER_W)], sem
    )
    c_tab.wait()
    c_idx.wait()

    t0a = tab_v[pl.ds(0, LANES)]
    t0b = tab_v[pl.ds(16, LANES)]
    t1a = tab_v[pl.ds(32, LANES)]
    t1b = tab_v[pl.ds(48, LANES)]
    t2a = tab_v[pl.ds(64, LANES)]
    t2b = tab_v[pl.ds(80, LANES)]
    zero16 = jnp.zeros((LANES,), jnp.int32)

    half = B_PER_W // 2
    copies = []
    for h in range(2):

        @plsc.parallel_loop(h * half, (h + 1) * half, unroll=8)
        def _row(i):
            vi = idx_v[pl.ds(i, LANES)]
            s = vi.at[zero16].get(mode="promise_in_bounds")
            m1 = s == 1
            m2 = s == 2
            out_v[i, pl.ds(0, LANES)] = jnp.where(m1, t1a, jnp.where(m2, t2a, t0a))
            out_v[i, pl.ds(LANES, LANES)] = jnp.where(
                m1, t1b, jnp.where(m2, t2b, t0b)
            )

        copies.append(
            pltpu.async_copy(
                out_v.at[pl.ds(h * half, half)],
                out_hbm.at[pl.ds(wid * B_PER_W + h * half, half)],
                sem,
            )
        )
    for c in copies:
        c.wait()


@jax.jit
def _lookup(idx_flat, table_flat):
    mesh = plsc.VectorSubcoreMesh(core_axis_name="c", subcore_axis_name="s")
    return pl.kernel(
        _lookup_body,
        out_type=jax.ShapeDtypeStruct((BATCH, EMBD_DIM), jnp.float32),
        mesh=mesh,
        compiler_params=pltpu.CompilerParams(needs_layout_passes=False),
        scratch_types=[
            pltpu.VMEM((3 * EMBD_DIM,), jnp.float32),
            pltpu.VMEM((B_PER_W, EMBD_DIM), jnp.float32),
            pltpu.VMEM((B_PER_W + LANES,), jnp.int32),
            pltpu.SemaphoreType.DMA,
        ],
    )(idx_flat, table_flat)


def kernel(bus_type, embd_table):
    idx_flat = bus_type.astype(jnp.int32).reshape(BATCH)
    return _lookup(idx_flat, embd_table.reshape(-1))
